# Initial kernel scaffold; baseline (speedup 1.0000x reference)
import jax
import jax.numpy as jnp
from jax.experimental import pallas as pl


def kernel(x, codebook):
    raise NotImplementedError("write your pallas kernel here")


# speedup vs baseline: 2.2962x; 2.2962x over previous
import functools

import jax
import jax.numpy as jnp
from jax.experimental import pallas as pl

CODEBOOK_SIZE = 512
DIM = 32
BETA = 0.25


def _vq_kernel(x_ref, cb_ref, quant_ref, idx_ref, loss_ref):
    b = pl.program_id(0)
    xb = x_ref[0]
    cb = cb_ref[...]
    m = jax.lax.dot_general(
        cb, xb, (((1,), (0,)), ((), ())),
        precision=jax.lax.Precision.DEFAULT,
        preferred_element_type=jnp.float32,
    )
    a = jnp.sum(xb * xb, axis=0, keepdims=True)
    c2 = jnp.sum(cb * cb, axis=1)[:, None]
    dist = (a - 2.0 * m) + c2
    iota = jax.lax.broadcasted_iota(jnp.int32, dist.shape, 0)
    dmin = jnp.min(dist, axis=0, keepdims=True)
    idx = jnp.min(jnp.where(dist == dmin, iota, CODEBOOK_SIZE),
                  axis=0)
    idx_ref[0, 0, :] = idx
    onehot = (iota == idx[None, :]).astype(jnp.float32)
    quant = jax.lax.dot_general(
        cb, onehot, (((0,), (0,)), ((), ())),
        precision=jax.lax.Precision.HIGHEST,
        preferred_element_type=jnp.float32,
    )
    quant_ref[0] = quant
    diff = quant - xb
    part = jnp.sum(diff * diff).reshape(1, 1)
    @pl.when(b == 0)
    def _():
        loss_ref[...] = jnp.zeros_like(loss_ref)
    loss_ref[...] += part


@functools.partial(jax.jit, static_argnames=())
def kernel(x, codebook):
    bsz, dim, tlen = x.shape
    quant, idx3, loss_sum = pl.pallas_call(
        _vq_kernel,
        grid=(bsz,),
        in_specs=[
            pl.BlockSpec((1, dim, tlen), lambda b: (b, 0, 0)),
            pl.BlockSpec((CODEBOOK_SIZE, dim), lambda b: (0, 0)),
        ],
        out_specs=[
            pl.BlockSpec((1, dim, tlen), lambda b: (b, 0, 0)),
            pl.BlockSpec((1, 1, tlen), lambda b: (b, 0, 0)),
            pl.BlockSpec((1, 1), lambda b: (0, 0)),
        ],
        out_shape=[
            jax.ShapeDtypeStruct((bsz, dim, tlen), jnp.float32),
            jax.ShapeDtypeStruct((bsz, 1, tlen), jnp.int32),
            jax.ShapeDtypeStruct((1, 1), jnp.float32),
        ],
    )(x, codebook)
    idx = idx3.reshape(bsz, tlen)
    loss_vq = loss_sum[0, 0] / jnp.float32(bsz * dim * tlen)
    loss_commit = jnp.float32(BETA) * loss_vq
    return (quant, idx, loss_vq, loss_commit)
